# baseline (speedup 1.0000x reference)
import jax
import jax.numpy as jnp
from jax import lax
from jax.experimental import pallas as pl
from jax.experimental.pallas import tpu as pltpu
from jax.experimental.pallas import tpu_sc as plsc

N = 1048576
NC, NS = 2, 16
NW = NC * NS
BPW = N // NW
CHUNK = 512
GB = 128
K = CHUNK // GB
NCHUNK = BPW // CHUNK


def _sc_body(c0_hbm, c1_hbm, c3_hbm, c4_hbm, c5_hbm, idx_hbm,
             r0_hbm, r1_hbm, vx_hbm, vy_hbm, vz_hbm,
             idx_v, ga, gb, gsem_a, gsem_b):
    c = lax.axis_index("c")
    s = lax.axis_index("s")
    wid = s * NC + c
    pltpu.sync_copy(idx_hbm.at[pl.ds(wid * (BPW // GB), BPW // GB)], idx_v)

    cols_in = (c0_hbm, c1_hbm, c3_hbm, c4_hbm, c5_hbm)
    outs = (r0_hbm, r1_hbm, vx_hbm, vy_hbm, vz_hbm)

    def fire(ci, buf, sem):
        for j in range(K):
            row = idx_v.at[ci * K + j]
            for q in range(5):
                pltpu.async_copy(
                    cols_in[q].at[row],
                    buf.at[pl.ds(q * CHUNK + j * GB, GB)],
                    sem,
                )

    def drain(buf, sem):
        pltpu.make_async_copy(
            c0_hbm.at[pl.ds(0, 5 * CHUNK)], buf, sem
        ).wait()

    def process(ci, buf):
        def group(g, carry2):
            r3 = buf[pl.ds(2 * CHUNK + g * 16, 16)]
            r4 = buf[pl.ds(3 * CHUNK + g * 16, 16)]
            r5 = buf[pl.ds(4 * CHUNK + g * 16, 16)]

            ssq = r3 * r3 + r4 * r4 + r5 * r5
            seed = plsc.bitcast(
                jnp.int32(0x5F3759DF) - lax.shift_right_logical(
                    plsc.bitcast(ssq, jnp.int32), 1
                ),
                jnp.float32,
            )
            half = 0.5 * ssq
            y = seed * (1.5 - half * seed * seed)
            y = y * (1.5 - half * y * y)
            y = y * (1.5 - half * y * y)
            inv = y

            buf[pl.ds(2 * CHUNK + g * 16, 16)] = r3 * inv
            buf[pl.ds(3 * CHUNK + g * 16, 16)] = r4 * inv
            buf[pl.ds(4 * CHUNK + g * 16, 16)] = r5 * inv
            return carry2

        lax.fori_loop(0, CHUNK // 16, group, 0)

        base = wid * BPW + ci * CHUNK
        for q in range(5):
            pltpu.sync_copy(
                buf.at[pl.ds(q * CHUNK, CHUNK)], outs[q].at[pl.ds(base, CHUNK)]
            )

    bufs_a = ga
    bufs_b = gb

    fire(0, bufs_a, gsem_a)

    def body(tt, carry):
        ca = 2 * tt
        fire(ca + 1, bufs_b, gsem_b)
        drain(bufs_a, gsem_a)
        process(ca, bufs_a)
        fire(ca + 2, bufs_a, gsem_a)
        drain(bufs_b, gsem_b)
        process(ca + 1, bufs_b)
        return carry

    lax.fori_loop(0, NCHUNK // 2 - 1, body, 0)

    fire(NCHUNK - 1, bufs_b, gsem_b)
    drain(bufs_a, gsem_a)
    process(NCHUNK - 2, bufs_a)
    drain(bufs_b, gsem_b)
    process(NCHUNK - 1, bufs_b)


_sc_call = pl.kernel(
    _sc_body,
    out_type=tuple(
        jax.ShapeDtypeStruct((N,), jnp.float32) for _ in range(5)
    ),
    mesh=plsc.VectorSubcoreMesh(core_axis_name="c", subcore_axis_name="s"),
    compiler_params=pltpu.CompilerParams(
        needs_layout_passes=False, use_tc_tiling_on_sc=False
    ),
    scratch_types=[
        pltpu.VMEM((BPW // GB, GB), jnp.int32),
        pltpu.VMEM((5 * CHUNK,), jnp.float32),
        pltpu.VMEM((5 * CHUNK,), jnp.float32),
        pltpu.SemaphoreType.DMA,
        pltpu.SemaphoreType.DMA,
    ],
)


def kernel(all_rays, indices):
    cols = [all_rays[:, c] for c in (0, 1, 3, 4, 5)]
    idx2 = indices.reshape(N // GB, GB)
    r0g, r1g, vx, vy, vz = _sc_call(*cols, idx2)
    p = jnp.stack(
        [jnp.zeros((N,), jnp.float32), 1000.0 * r0g, 1000.0 * r1g], axis=0
    ).T
    v = jnp.stack([-vz, vx, vy], axis=0).T
    return (p, v)


# speedup vs baseline: 1.0152x; 1.0152x over previous
import jax
import jax.numpy as jnp
from jax import lax
from jax.experimental import pallas as pl
from jax.experimental.pallas import tpu as pltpu
from jax.experimental.pallas import tpu_sc as plsc

N = 1048576
NC, NS = 2, 16
NW = NC * NS
BPW = N // NW
CHUNK = 1024
GB = 128
K = CHUNK // GB
NCHUNK = BPW // CHUNK


def _sc_body(c0_hbm, c1_hbm, c3_hbm, c4_hbm, c5_hbm, idx_hbm,
             r0_hbm, r1_hbm, vx_hbm, vy_hbm, vz_hbm,
             idx_v, ga, gb, gsem_a, gsem_b):
    c = lax.axis_index("c")
    s = lax.axis_index("s")
    wid = s * NC + c
    pltpu.sync_copy(idx_hbm.at[pl.ds(wid * (BPW // GB), BPW // GB)], idx_v)

    cols_in = (c0_hbm, c1_hbm, c3_hbm, c4_hbm, c5_hbm)
    outs = (r0_hbm, r1_hbm, vx_hbm, vy_hbm, vz_hbm)

    def fire(ci, buf, sem):
        for j in range(K):
            row = idx_v.at[ci * K + j]
            for q in range(5):
                pltpu.async_copy(
                    cols_in[q].at[row],
                    buf.at[pl.ds(q * CHUNK + j * GB, GB)],
                    sem,
                )

    def drain(buf, sem):
        pltpu.make_async_copy(
            c0_hbm.at[pl.ds(0, 5 * CHUNK)], buf, sem
        ).wait()

    def process(ci, buf):
        def group(g, carry2):
            r3 = buf[pl.ds(2 * CHUNK + g * 16, 16)]
            r4 = buf[pl.ds(3 * CHUNK + g * 16, 16)]
            r5 = buf[pl.ds(4 * CHUNK + g * 16, 16)]

            ssq = r3 * r3 + r4 * r4 + r5 * r5
            seed = plsc.bitcast(
                jnp.int32(0x5F3759DF) - lax.shift_right_logical(
                    plsc.bitcast(ssq, jnp.int32), 1
                ),
                jnp.float32,
            )
            half = 0.5 * ssq
            y = seed * (1.5 - half * seed * seed)
            y = y * (1.5 - half * y * y)
            y = y * (1.5 - half * y * y)
            inv = y

            buf[pl.ds(2 * CHUNK + g * 16, 16)] = r3 * inv
            buf[pl.ds(3 * CHUNK + g * 16, 16)] = r4 * inv
            buf[pl.ds(4 * CHUNK + g * 16, 16)] = r5 * inv
            return carry2

        lax.fori_loop(0, CHUNK // 16, group, 0)

        base = wid * BPW + ci * CHUNK
        for q in range(5):
            pltpu.sync_copy(
                buf.at[pl.ds(q * CHUNK, CHUNK)], outs[q].at[pl.ds(base, CHUNK)]
            )

    bufs_a = ga
    bufs_b = gb

    fire(0, bufs_a, gsem_a)

    def body(tt, carry):
        ca = 2 * tt
        fire(ca + 1, bufs_b, gsem_b)
        drain(bufs_a, gsem_a)
        process(ca, bufs_a)
        fire(ca + 2, bufs_a, gsem_a)
        drain(bufs_b, gsem_b)
        process(ca + 1, bufs_b)
        return carry

    lax.fori_loop(0, NCHUNK // 2 - 1, body, 0)

    fire(NCHUNK - 1, bufs_b, gsem_b)
    drain(bufs_a, gsem_a)
    process(NCHUNK - 2, bufs_a)
    drain(bufs_b, gsem_b)
    process(NCHUNK - 1, bufs_b)


_sc_call = pl.kernel(
    _sc_body,
    out_type=tuple(
        jax.ShapeDtypeStruct((N,), jnp.float32) for _ in range(5)
    ),
    mesh=plsc.VectorSubcoreMesh(core_axis_name="c", subcore_axis_name="s"),
    compiler_params=pltpu.CompilerParams(
        needs_layout_passes=False, use_tc_tiling_on_sc=False
    ),
    scratch_types=[
        pltpu.VMEM((BPW // GB, GB), jnp.int32),
        pltpu.VMEM((5 * CHUNK,), jnp.float32),
        pltpu.VMEM((5 * CHUNK,), jnp.float32),
        pltpu.SemaphoreType.DMA,
        pltpu.SemaphoreType.DMA,
    ],
)


def kernel(all_rays, indices):
    cols = [all_rays[:, c] for c in (0, 1, 3, 4, 5)]
    idx2 = indices.reshape(N // GB, GB)
    r0g, r1g, vx, vy, vz = _sc_call(*cols, idx2)
    p = jnp.stack(
        [jnp.zeros((N,), jnp.float32), 1000.0 * r0g, 1000.0 * r1g], axis=0
    ).T
    v = jnp.stack([-vz, vx, vy], axis=0).T
    return (p, v)
